# baseline (speedup 1.0000x reference)
import functools

import jax
import jax.numpy as jnp
from jax import lax
from jax.experimental import pallas as pl
from jax.experimental.pallas import tpu as pltpu
from jax.experimental.pallas import tpu_sc as plsc

N = 8192
L = 64
P = L * L
CB = 256
NCOPY = N // CB
S = 8

_INFO = plsc.get_sparse_core_info()
_NC, _NS = _INFO.num_cores, _INFO.num_subcores
NW = _NC * _NS
ROWS_PER_W = N // NW
NVEC = ROWS_PER_W // 16


def _fill_body(out_ref, tbuf, sems):
    flat = lax.broadcasted_iota(jnp.int32, (CB, P), 1)
    tbuf[...] = jnp.where(flat == 0, jnp.float32(1.0), jnp.float32(0.0))
    copies = [
        pltpu.make_async_copy(
            tbuf, out_ref.at[pl.ds(i * CB, CB), :], sems.at[i % S])
        for i in range(NCOPY)
    ]
    for i in range(S):
        copies[i].start()
    for i in range(NCOPY - S):
        copies[i].wait()
        copies[i + S].start()
    for i in range(NCOPY - S, NCOPY):
        copies[i].wait()


_tc_fill = pl.pallas_call(
    _fill_body,
    grid=(1,),
    in_specs=[],
    out_specs=pl.BlockSpec(memory_space=pl.ANY),
    out_shape=jax.ShapeDtypeStruct((N, P), jnp.float32),
    scratch_shapes=[
        pltpu.VMEM((CB, P), jnp.float32),
        pltpu.SemaphoreType.DMA((S,)),
    ],
)


@functools.partial(
    pl.kernel,
    mesh=plsc.VectorSubcoreMesh(core_axis_name="c", subcore_axis_name="s"),
    out_type=(),
    scratch_types=[
        pltpu.VMEM((ROWS_PER_W,), jnp.int32),
        pltpu.VMEM((ROWS_PER_W,), jnp.int32),
        pltpu.VMEM((2, 128), jnp.int32),
        pltpu.VMEM((2, 128), jnp.float32),
    ],
)
def _sc_poke(out_hbm, y_hbm, x_hbm, y_v, x_v, idx_v, val_v):
    base_row = (lax.axis_index("s") * _NC + lax.axis_index("c")) * ROWS_PER_W
    pltpu.sync_copy(y_hbm.at[pl.ds(base_row, ROWS_PER_W)], y_v)
    pltpu.sync_copy(x_hbm.at[pl.ds(base_row, ROWS_PER_W)], x_v)

    iota16 = lax.iota(jnp.int32, 16)
    neg1 = jnp.full((16,), -1.0, jnp.float32)
    for cc in range(NVEC):
        yv = y_v[pl.ds(cc * 16, 16)]
        xv = x_v[pl.ds(cc * 16, 16)]
        sep_idx = (base_row + cc * 16 + iota16) * P + yv * L + xv
        j, k = divmod(cc, 8)
        idx_v[j, pl.ds(k * 16, 16)] = sep_idx
        val_v[j, pl.ds(k * 16, 16)] = neg1
    for j in range(2):
        pltpu.sync_copy(val_v.at[j], out_hbm.at[idx_v.at[j]])


def kernel(lats, x_seps, y_seps):
    y = y_seps.astype(jnp.int32)
    x = x_seps.astype(jnp.int32)
    filled = _tc_fill().reshape(N * P)
    ref = jax.new_ref(filled)
    _sc_poke(ref, y, x)
    return ref[...].reshape(N, 1, L, L)
